# baseline (speedup 1.0000x reference)
import functools

import jax
import jax.numpy as jnp
from jax import lax
from jax.experimental import pallas as pl
from jax.experimental.pallas import tpu as pltpu
from jax.experimental.pallas import tpu_sc as plsc

_NC, _NS, _L = 2, 16, 16
_NW = _NC * _NS
_CS = 32


def _body(x_hbm, pe_hbm, out_hbm, pe_v, x_v):
    H = 1024
    n_chunk = (x_hbm.shape[1] // H) // (_NW * _CS)
    B = x_hbm.shape[0]
    wid = lax.axis_index("s") * _NC + lax.axis_index("c")
    words = _CS * H

    for c in range(n_chunk):
        base = (wid * n_chunk + c) * words
        pltpu.sync_copy(pe_hbm.at[pl.ds(base, words)], pe_v)
        for b in range(B):
            pltpu.sync_copy(x_hbm.at[b, pl.ds(base, words)], x_v)

            @plsc.parallel_loop(0, words // _L, unroll=8)
            def _add(j):
                off = j * _L
                x_v[pl.ds(off, _L)] = x_v[pl.ds(off, _L)] + pe_v[pl.ds(off, _L)]

            pltpu.sync_copy(x_v, out_hbm.at[b, pl.ds(base, words)])


def kernel(x, pe):
    B, S, H = x.shape
    x2 = x.reshape(B, S * H)
    pe2 = pe[:S].reshape(S * H)
    run = pl.kernel(
        _body,
        out_type=jax.ShapeDtypeStruct((B, S * H), x.dtype),
        mesh=plsc.VectorSubcoreMesh(core_axis_name="c", subcore_axis_name="s"),
        scratch_types=[
            pltpu.VMEM((_CS * H,), jnp.float32),
            pltpu.VMEM((_CS * H,), jnp.float32),
        ],
    )
    return run(x2, pe2).reshape(B, S, H)


# speedup vs baseline: 1.2417x; 1.2417x over previous
import jax
import jax.numpy as jnp
from jax import lax
from jax.experimental import pallas as pl
from jax.experimental.pallas import tpu as pltpu
from jax.experimental.pallas import tpu_sc as plsc

_NC, _NS, _L = 2, 16, 16
_NW = _NC * _NS
_CS = 16
_NBUF = 3
_H = 1024


def _body(x_hbm, pe_hbm, out_hbm,
          pe_v0, pe_v1, xv0, xv1, xv2,
          in_s0, in_s1, in_s2, out_s0, out_s1, out_s2, pe_s0, pe_s1):
    B = x_hbm.shape[0]
    n_chunk = x_hbm.shape[1] // (_NW * _CS * _H)
    n_items = n_chunk * B
    words = _CS * _H
    wid = lax.axis_index("s") * _NC + lax.axis_index("c")
    w_base = wid * n_chunk * words

    pe_v = [pe_v0, pe_v1]
    xv = [xv0, xv1, xv2]
    in_s = [in_s0, in_s1, in_s2]
    out_s = [out_s0, out_s1, out_s2]
    pe_s = [pe_s0, pe_s1]

    def start_in(i):
        c, b = divmod(i, B)
        return pltpu.async_copy(
            x_hbm.at[b, pl.ds(w_base + c * words, words)],
            xv[i % _NBUF], in_s[i % _NBUF])

    def start_pe(c):
        return pltpu.async_copy(
            pe_hbm.at[pl.ds(w_base + c * words, words)],
            pe_v[c % 2], pe_s[c % 2])

    pe_cp = {0: start_pe(0)}
    in_cp = {0: start_in(0)}
    out_cp = {}

    for i in range(n_items):
        c, b = divmod(i, B)
        if i + 1 < n_items:
            if i - (_NBUF - 1) in out_cp:
                out_cp[i - (_NBUF - 1)].wait()
            in_cp[i + 1] = start_in(i + 1)
        if b == 0 and c + 1 < n_chunk:
            pe_cp[c + 1] = start_pe(c + 1)
        in_cp[i].wait()
        if b == 0:
            pe_cp[c].wait()
        dst = xv[i % _NBUF]
        src = pe_v[c % 2]

        @plsc.parallel_loop(0, words // _L, unroll=8)
        def _add(j):
            off = j * _L
            dst[pl.ds(off, _L)] = dst[pl.ds(off, _L)] + src[pl.ds(off, _L)]

        out_cp[i] = pltpu.async_copy(
            dst, out_hbm.at[b, pl.ds(w_base + c * words, words)],
            out_s[i % _NBUF])

    for i in range(max(0, n_items - _NBUF), n_items):
        out_cp[i].wait()


def kernel(x, pe):
    B, S, H = x.shape
    x2 = x.reshape(B, S * H)
    pe2 = pe[:S].reshape(S * H)
    run = pl.kernel(
        _body,
        out_type=jax.ShapeDtypeStruct((B, S * H), x.dtype),
        mesh=plsc.VectorSubcoreMesh(core_axis_name="c", subcore_axis_name="s"),
        scratch_types=(
            [pltpu.VMEM((_CS * _H,), jnp.float32)] * 2
            + [pltpu.VMEM((_CS * _H,), jnp.float32)] * _NBUF
            + [pltpu.SemaphoreType.DMA] * (2 * _NBUF + 2)
        ),
    )
    return run(x2, pe2).reshape(B, S, H)


# speedup vs baseline: 3.2191x; 2.5925x over previous
import jax
import jax.numpy as jnp
from jax import lax
from jax.experimental import pallas as pl
from jax.experimental.pallas import tpu as pltpu
from jax.experimental.pallas import tpu_sc as plsc

_NC, _NS, _L = 2, 16, 16
_NW = _NC * _NS
_CS = 16
_NBUF = 3


def _body(x_hbm, pe_hbm, out_hbm,
          pe_v0, pe_v1, xv0, xv1, xv2,
          in_s0, in_s1, in_s2, out_s0, out_s1, out_s2, pe_s0, pe_s1):
    B, S, H = x_hbm.shape
    n_chunk = S // (_NW * _CS)
    n_items = n_chunk * B
    n_vec = _CS * H // _L
    row_shift = (H // _L).bit_length() - 1
    wid = lax.axis_index("s") * _NC + lax.axis_index("c")
    w_row = wid * n_chunk * _CS

    pe_v = [pe_v0, pe_v1]
    xv = [xv0, xv1, xv2]
    in_s = [in_s0, in_s1, in_s2]
    out_s = [out_s0, out_s1, out_s2]
    pe_s = [pe_s0, pe_s1]

    def start_in(i):
        c, b = divmod(i, B)
        return pltpu.async_copy(
            x_hbm.at[b, pl.ds(w_row + c * _CS, _CS), :],
            xv[i % _NBUF], in_s[i % _NBUF])

    def start_pe(c):
        return pltpu.async_copy(
            pe_hbm.at[pl.ds(w_row + c * _CS, _CS), :],
            pe_v[c % 2], pe_s[c % 2])

    pe_cp = {0: start_pe(0)}
    in_cp = {0: start_in(0)}
    out_cp = {}

    for i in range(n_items):
        c, b = divmod(i, B)
        if i + 1 < n_items:
            if i - (_NBUF - 1) in out_cp:
                out_cp[i - (_NBUF - 1)].wait()
            in_cp[i + 1] = start_in(i + 1)
        if b == 0 and c + 1 < n_chunk:
            pe_cp[c + 1] = start_pe(c + 1)
        in_cp[i].wait()
        if b == 0:
            pe_cp[c].wait()
        dst = xv[i % _NBUF]
        src = pe_v[c % 2]

        @plsc.parallel_loop(0, n_vec, unroll=8)
        def _add(j):
            r = lax.shift_right_logical(j, row_shift)
            col = pl.multiple_of(lax.shift_left(jnp.bitwise_and(j, (H // _L) - 1), 4), _L)
            dst[r, pl.ds(col, _L)] = dst[r, pl.ds(col, _L)] + src[r, pl.ds(col, _L)]

        out_cp[i] = pltpu.async_copy(
            dst, out_hbm.at[b, pl.ds(w_row + c * _CS, _CS), :],
            out_s[i % _NBUF])

    for i in range(max(0, n_items - _NBUF), n_items):
        out_cp[i].wait()


def kernel(x, pe):
    B, S, H = x.shape
    if pe.shape[0] != S:
        pe = pe[:S]
    run = pl.kernel(
        _body,
        out_type=jax.ShapeDtypeStruct((B, S, H), x.dtype),
        mesh=plsc.VectorSubcoreMesh(core_axis_name="c", subcore_axis_name="s"),
        scratch_types=(
            [pltpu.VMEM((_CS, H), jnp.float32)] * 2
            + [pltpu.VMEM((_CS, H), jnp.float32)] * _NBUF
            + [pltpu.SemaphoreType.DMA] * (2 * _NBUF + 2)
        ),
    )
    return run(x, pe)
